# baseline (speedup 1.0000x reference)
import functools

import jax
import jax.numpy as jnp
from jax import lax
from jax.experimental import pallas as pl
from jax.experimental.pallas import tpu as pltpu
from jax.experimental.pallas import tpu_sc as plsc

_C = 20
_K = 10
_MAXD = _C * _K
_IMG = 512.0
_SCORE_THR = 0.3
_IOU_THR = 0.5
_CAP = 2048
_CH = 2048


def _sp(x, dt):
    return jnp.zeros((16,), dt) + x


def _sc_nms(x1h, y1h, x2h, y2h, sch, clh):
    B, NPAD = sch.shape
    SENT = NPAD
    NR = NPAD + 16
    NCH = NPAD // _CH
    mesh = plsc.VectorSubcoreMesh(core_axis_name="c", subcore_axis_name="s",
                                  num_cores=2, num_subcores=16)

    @functools.partial(
        pl.kernel,
        out_type=jax.ShapeDtypeStruct((B, 2, 6 * 16 * _K), jnp.float32),
        mesh=mesh,
        compiler_params=pltpu.CompilerParams(needs_layout_passes=False),
        scratch_types=[
            pltpu.VMEM((NR,), jnp.float32),
            pltpu.VMEM((NR,), jnp.float32),
            pltpu.VMEM((NR,), jnp.float32),
            pltpu.VMEM((NR,), jnp.float32),
            pltpu.VMEM((NR,), jnp.float32),
            pltpu.VMEM((_CAP * _K + 16,), jnp.int32),
            pltpu.VMEM((_CH,), jnp.int32),
            pltpu.VMEM((6 * 16 * _K,), jnp.float32),
            pltpu.VMEM((32,), jnp.int32),
        ],
    )
    def k(x1_h, y1_h, x2_h, y2_h, sc_h, cl_h, out_h,
          X1, Y1, X2, Y2, SCO, LST, CLS, STG, CNT):
        b = lax.axis_index("s")
        half = lax.axis_index("c")
        cbase = half * _K
        iota16 = lax.iota(jnp.int32, 16)

        @pl.when(b < B)
        def _():
            pltpu.sync_copy(x1_h.at[b], X1.at[pl.ds(0, NPAD)])
            pltpu.sync_copy(y1_h.at[b], Y1.at[pl.ds(0, NPAD)])
            pltpu.sync_copy(x2_h.at[b], X2.at[pl.ds(0, NPAD)])
            pltpu.sync_copy(y2_h.at[b], Y2.at[pl.ds(0, NPAD)])
            pltpu.sync_copy(sc_h.at[b], SCO.at[pl.ds(0, NPAD)])
            zf = jnp.zeros((16,), jnp.float32)
            X1[pl.ds(NPAD, 16)] = zf
            Y1[pl.ds(NPAD, 16)] = zf
            X2[pl.ds(NPAD, 16)] = zf
            Y2[pl.ds(NPAD, 16)] = zf
            SCO[pl.ds(NPAD, 16)] = zf

            def clipb(i, _):
                o = i * 16
                X1[pl.ds(o, 16)] = jnp.clip(X1[pl.ds(o, 16)], 0.0, _IMG)
                Y1[pl.ds(o, 16)] = jnp.clip(Y1[pl.ds(o, 16)], 0.0, _IMG)
                X2[pl.ds(o, 16)] = jnp.clip(X2[pl.ds(o, 16)], 0.0, _IMG)
                Y2[pl.ds(o, 16)] = jnp.clip(Y2[pl.ds(o, 16)], 0.0, _IMG)
                return 0
            lax.fori_loop(0, NPAD // 16, clipb, 0)

            sentv = _sp(SENT, jnp.int32)
            def fillb(i, _):
                LST[pl.ds(i * 16, 16)] = sentv
                return 0
            lax.fori_loop(0, (_CAP * _K) // 16, fillb, 0)

            CNT[pl.ds(0, 16)] = jnp.zeros((16,), jnp.int32)
            CNT[pl.ds(16, 16)] = jnp.zeros((16,), jnp.int32)
            dumpv = _sp(_CAP * _K, jnp.int32) + iota16

            def chunkb(ch, _):
                pltpu.sync_copy(cl_h.at[b, pl.ds(ch * _CH, _CH)], CLS)

                def vb(v, _):
                    o = v * 16
                    go = ch * _CH + o
                    cl = CLS[pl.ds(o, 16)]
                    sv = SCO[pl.ds(go, 16)]
                    wv = X2[pl.ds(go, 16)] - X1[pl.ds(go, 16)]
                    hv = Y2[pl.ds(go, 16)] - Y1[pl.ds(go, 16)]
                    jl = cl - cbase
                    valid = (sv > _SCORE_THR) & (wv >= 0.01) & (hv >= 0.01) \
                        & (jl >= 0) & (jl < _K)
                    xkey = jnp.where(valid, jl, _K + 10 + iota16)
                    cnt16, lastm = plsc.scan_count(xkey)
                    basev = plsc.load_gather(CNT, [jnp.where(valid, jl, 15)])
                    posn = basev + cnt16 - 1
                    ok = valid & (posn < _CAP)
                    tgt = jnp.where(ok, jl * _CAP + posn, dumpv)
                    gi = _sp(go, jnp.int32) + iota16
                    plsc.store_scatter(LST, [tgt], gi)
                    contrib = ok & lastm
                    plsc.addupdate_scatter(
                        CNT, [jnp.where(contrib, jl, 15)],
                        jnp.where(contrib, cnt16, 0))
                    return 0

                return lax.fori_loop(0, _CH // 16, vb, 0)

            lax.fori_loop(0, NCH, chunkb, 0)

            BIGP = jnp.int32(2 ** 30)
            for j in range(_K):
                base = j * _CAP
                cnt = jnp.minimum(CNT[pl.ds(j, 16)][0], _CAP)
                nv = (cnt + 15) // 16

                def roundb(r, kc):
                    KX1, KY1, KX2, KY2, KS, KM = kc

                    def amax(i, mc):
                        mv, pv = mc
                        il = LST[pl.ds(base + i * 16, 16)]
                        sv = plsc.load_gather(SCO, [il])
                        curpos = _sp(i * 16, jnp.int32) + iota16
                        gt = sv > mv
                        pv = jnp.where(gt, curpos, pv)
                        mv = jnp.where(gt, sv, mv)
                        return (mv, pv)

                    mv, pv = lax.fori_loop(
                        0, nv, amax,
                        (_sp(-1.0, jnp.float32), _sp(BIGP, jnp.int32)))
                    m = jnp.max(mv)
                    alive = m > _SCORE_THR
                    pos = jnp.min(jnp.where(mv == m, pv, BIGP))
                    safe = jnp.where(pos >= BIGP, 0, pos)
                    oi = plsc.load_gather(LST, [_sp(base, jnp.int32) +
                                                _sp(safe, jnp.int32)])
                    bx1 = plsc.load_gather(X1, [oi])
                    by1 = plsc.load_gather(Y1, [oi])
                    bx2 = plsc.load_gather(X2, [oi])
                    by2 = plsc.load_gather(Y2, [oi])
                    barea = (bx2 - bx1) * (by2 - by1)

                    def suppb(i, _):
                        sl = pl.ds(base + i * 16, 16)
                        il = LST[sl]
                        cx1 = plsc.load_gather(X1, [il])
                        cy1 = plsc.load_gather(Y1, [il])
                        cx2 = plsc.load_gather(X2, [il])
                        cy2 = plsc.load_gather(Y2, [il])
                        xx1 = jnp.maximum(bx1, cx1)
                        yy1 = jnp.maximum(by1, cy1)
                        xx2 = jnp.minimum(bx2, cx2)
                        yy2 = jnp.minimum(by2, cy2)
                        inter = jnp.maximum(xx2 - xx1, 0.0) * \
                            jnp.maximum(yy2 - yy1, 0.0)
                        carea = (cx2 - cx1) * (cy2 - cy1)
                        union = barea + carea - inter
                        iou = inter / jnp.maximum(union, 1e-9)
                        LST[sl] = jnp.where(iou > _IOU_THR, sentv, il)
                        return 0
                    lax.fori_loop(0, nv, suppb, 0)

                    sel = (iota16 == r) & alive
                    KX1 = jnp.where(sel, bx1, KX1)
                    KY1 = jnp.where(sel, by1, KY1)
                    KX2 = jnp.where(sel, bx2, KX2)
                    KY2 = jnp.where(sel, by2, KY2)
                    KS = jnp.where(sel, _sp(m, jnp.float32), KS)
                    KM = jnp.where(sel, 1.0, KM)
                    return (KX1, KY1, KX2, KY2, KS, KM)

                z = jnp.zeros((16,), jnp.float32)
                KX1, KY1, KX2, KY2, KS, KM = lax.fori_loop(
                    0, _K, roundb, (z, z, z, z, z, z))
                sb = j * 96
                STG[pl.ds(sb + 0, 16)] = KX1
                STG[pl.ds(sb + 16, 16)] = KY1
                STG[pl.ds(sb + 32, 16)] = KX2
                STG[pl.ds(sb + 48, 16)] = KY2
                STG[pl.ds(sb + 64, 16)] = KS
                STG[pl.ds(sb + 80, 16)] = KM

            pltpu.sync_copy(STG, out_h.at[b, half])

    return k(x1h, y1h, x2h, y2h, sch, clh)


def _pack_body(vr_ref, vc_ref, o_ref):
    vr = vr_ref[0]
    vc = vc_ref[0]
    Mrow = vr[5:6, :]
    ii = lax.broadcasted_iota(jnp.int32, (_MAXD, _MAXD), 0)
    jj = lax.broadcasted_iota(jnp.int32, (_MAXD, _MAXD), 1)
    triu = (ii <= jj).astype(jnp.float32)
    posrow = jnp.dot(Mrow, triu, precision=lax.Precision.HIGHEST,
                     preferred_element_type=jnp.float32)
    posi = posrow.astype(jnp.int32) - 1
    oh = ((ii == posi) & (Mrow > 0.5)).astype(jnp.float32)
    out6 = jnp.dot(oh, vc, precision=lax.Precision.HIGHEST,
                   preferred_element_type=jnp.float32)
    o_ref[0] = out6[:, 0:5]


def kernel(pred_boxes, pred_scores, pred_cls_inds):
    B, N = pred_scores.shape
    NPAD = ((N + _CH - 1) // _CH) * _CH
    pz = ((0, 0), (0, NPAD - N))
    x1 = jnp.pad(pred_boxes[:, :, 0], pz)
    y1 = jnp.pad(pred_boxes[:, :, 1], pz)
    x2 = jnp.pad(pred_boxes[:, :, 2], pz)
    y2 = jnp.pad(pred_boxes[:, :, 3], pz)
    sc = jnp.pad(pred_scores, pz)
    cl = jnp.pad(pred_cls_inds, pz)
    kept = _sc_nms(x1, y1, x2, y2, sc, cl)
    k5 = kept.reshape(B, 2, _K, 6, 16)[:, :, :, :, :_K]
    vrow = k5.transpose(0, 3, 1, 2, 4).reshape(B, 6, _MAXD)
    vcol = vrow.transpose(0, 2, 1)
    out = pl.pallas_call(
        _pack_body,
        grid=(B,),
        in_specs=[
            pl.BlockSpec((1, 6, _MAXD), lambda b: (b, 0, 0)),
            pl.BlockSpec((1, _MAXD, 6), lambda b: (b, 0, 0)),
        ],
        out_specs=pl.BlockSpec((1, _MAXD, 5), lambda b: (b, 0, 0)),
        out_shape=jax.ShapeDtypeStruct((B, _MAXD, 5), jnp.float32),
    )(vrow, vcol)
    return out


# speedup vs baseline: 1.1086x; 1.1086x over previous
import functools

import jax
import jax.numpy as jnp
from jax import lax
from jax.experimental import pallas as pl
from jax.experimental.pallas import tpu as pltpu
from jax.experimental.pallas import tpu_sc as plsc

_C = 20
_K = 10
_MAXD = _C * _K
_IMG = 512.0
_SCORE_THR = 0.3
_IOU_THR = 0.5
_CAP = 2048
_CH = 2048


def _sp(x, dt):
    return jnp.zeros((16,), dt) + x


def _sc_nms(x1h, y1h, x2h, y2h, sch, clh):
    B, NPAD = sch.shape
    SENT = NPAD
    NR = NPAD + 16
    NCH = NPAD // _CH
    mesh = plsc.VectorSubcoreMesh(core_axis_name="c", subcore_axis_name="s",
                                  num_cores=2, num_subcores=16)

    @functools.partial(
        pl.kernel,
        out_type=jax.ShapeDtypeStruct((B, 2, 6 * 16 * _K), jnp.float32),
        mesh=mesh,
        compiler_params=pltpu.CompilerParams(needs_layout_passes=False),
        scratch_types=[
            pltpu.VMEM((NR,), jnp.float32),
            pltpu.VMEM((NR,), jnp.float32),
            pltpu.VMEM((NR,), jnp.float32),
            pltpu.VMEM((NR,), jnp.float32),
            pltpu.VMEM((NR,), jnp.float32),
            pltpu.VMEM((_CAP * _K + 16,), jnp.int32),
            pltpu.VMEM((_CH,), jnp.int32),
            pltpu.VMEM((6 * 16 * _K,), jnp.float32),
            pltpu.VMEM((32,), jnp.int32),
        ],
    )
    def k(x1_h, y1_h, x2_h, y2_h, sc_h, cl_h, out_h,
          X1, Y1, X2, Y2, SCO, LST, CLS, STG, CNT):
        b = lax.axis_index("s")
        half = lax.axis_index("c")
        cbase = half * _K
        iota16 = lax.iota(jnp.int32, 16)

        @pl.when(b < B)
        def _():
            pltpu.sync_copy(x1_h.at[b], X1.at[pl.ds(0, NPAD)])
            pltpu.sync_copy(y1_h.at[b], Y1.at[pl.ds(0, NPAD)])
            pltpu.sync_copy(x2_h.at[b], X2.at[pl.ds(0, NPAD)])
            pltpu.sync_copy(y2_h.at[b], Y2.at[pl.ds(0, NPAD)])
            pltpu.sync_copy(sc_h.at[b], SCO.at[pl.ds(0, NPAD)])
            zf = jnp.zeros((16,), jnp.float32)
            X1[pl.ds(NPAD, 16)] = zf
            Y1[pl.ds(NPAD, 16)] = zf
            X2[pl.ds(NPAD, 16)] = zf
            Y2[pl.ds(NPAD, 16)] = zf
            SCO[pl.ds(NPAD, 16)] = zf

            def clipb(i, _):
                o = i * 16
                X1[pl.ds(o, 16)] = jnp.clip(X1[pl.ds(o, 16)], 0.0, _IMG)
                Y1[pl.ds(o, 16)] = jnp.clip(Y1[pl.ds(o, 16)], 0.0, _IMG)
                X2[pl.ds(o, 16)] = jnp.clip(X2[pl.ds(o, 16)], 0.0, _IMG)
                Y2[pl.ds(o, 16)] = jnp.clip(Y2[pl.ds(o, 16)], 0.0, _IMG)
                return 0
            lax.fori_loop(0, NPAD // 16, clipb, 0)

            sentv = _sp(SENT, jnp.int32)
            def fillb(i, _):
                LST[pl.ds(i * 16, 16)] = sentv
                return 0
            lax.fori_loop(0, (_CAP * _K) // 16, fillb, 0)

            CNT[pl.ds(0, 16)] = jnp.zeros((16,), jnp.int32)
            CNT[pl.ds(16, 16)] = jnp.zeros((16,), jnp.int32)
            dumpv = _sp(_CAP * _K, jnp.int32) + iota16

            def chunkb(ch, _):
                pltpu.sync_copy(cl_h.at[b, pl.ds(ch * _CH, _CH)], CLS)

                def vb(v, _):
                    o = v * 16
                    go = ch * _CH + o
                    cl = CLS[pl.ds(o, 16)]
                    sv = SCO[pl.ds(go, 16)]
                    wv = X2[pl.ds(go, 16)] - X1[pl.ds(go, 16)]
                    hv = Y2[pl.ds(go, 16)] - Y1[pl.ds(go, 16)]
                    jl = cl - cbase
                    valid = (sv > _SCORE_THR) & (wv >= 0.01) & (hv >= 0.01) \
                        & (jl >= 0) & (jl < _K)
                    xkey = jnp.where(valid, jl, _K + 10 + iota16)
                    cnt16, lastm = plsc.scan_count(xkey)
                    basev = plsc.load_gather(CNT, [jnp.where(valid, jl, 15)])
                    posn = basev + cnt16 - 1
                    ok = valid & (posn < _CAP)
                    tgt = jnp.where(ok, jl * _CAP + posn, dumpv)
                    gi = _sp(go, jnp.int32) + iota16
                    plsc.store_scatter(LST, [tgt], gi)
                    contrib = ok & lastm
                    plsc.addupdate_scatter(
                        CNT, [jnp.where(contrib, jl, 15)],
                        jnp.where(contrib, cnt16, 0))
                    return 0

                return lax.fori_loop(0, _CH // 16, vb, 0)

            lax.fori_loop(0, NCH, chunkb, 0)

            BIGP = jnp.int32(2 ** 30)
            for j in range(_K):
                base = j * _CAP
                cnt = jnp.minimum(CNT[pl.ds(j, 16)][0], _CAP)
                nv4 = (cnt + 63) // 64

                def roundb(r, kc):
                    KX1, KY1, KX2, KY2, KS, KM = kc

                    def amax(i, mc):
                        acc = list(mc)
                        bi = base + i * 64
                        for u in range(4):
                            il = LST[pl.ds(bi + u * 16, 16)]
                            sv = plsc.load_gather(SCO, [il])
                            curpos = _sp(i * 64 + u * 16, jnp.int32) + iota16
                            mv, pv = acc[2 * u], acc[2 * u + 1]
                            gt = sv > mv
                            acc[2 * u] = jnp.where(gt, sv, mv)
                            acc[2 * u + 1] = jnp.where(gt, curpos, pv)
                        return tuple(acc)

                    mneg = _sp(-1.0, jnp.float32)
                    pbig = _sp(BIGP, jnp.int32)
                    acc = lax.fori_loop(
                        0, nv4, amax, (mneg, pbig) * 4)
                    mv01 = jnp.maximum(acc[0], acc[2])
                    mv23 = jnp.maximum(acc[4], acc[6])
                    mvall = jnp.maximum(mv01, mv23)
                    m = jnp.max(mvall)
                    alive = m > _SCORE_THR
                    pos = jnp.min(
                        jnp.minimum(
                            jnp.minimum(
                                jnp.where(acc[0] == m, acc[1], BIGP),
                                jnp.where(acc[2] == m, acc[3], BIGP)),
                            jnp.minimum(
                                jnp.where(acc[4] == m, acc[5], BIGP),
                                jnp.where(acc[6] == m, acc[7], BIGP))))
                    safe = jnp.where(pos >= BIGP, 0, pos)
                    oi = plsc.load_gather(LST, [_sp(base, jnp.int32) +
                                                _sp(safe, jnp.int32)])
                    bx1 = plsc.load_gather(X1, [oi])
                    by1 = plsc.load_gather(Y1, [oi])
                    bx2 = plsc.load_gather(X2, [oi])
                    by2 = plsc.load_gather(Y2, [oi])
                    barea = (bx2 - bx1) * (by2 - by1)

                    def suppb(i, _):
                        bi = base + i * 64
                        for u in range(4):
                            sl = pl.ds(bi + u * 16, 16)
                            il = LST[sl]
                            cx1 = plsc.load_gather(X1, [il])
                            cy1 = plsc.load_gather(Y1, [il])
                            cx2 = plsc.load_gather(X2, [il])
                            cy2 = plsc.load_gather(Y2, [il])
                            xx1 = jnp.maximum(bx1, cx1)
                            yy1 = jnp.maximum(by1, cy1)
                            xx2 = jnp.minimum(bx2, cx2)
                            yy2 = jnp.minimum(by2, cy2)
                            inter = jnp.maximum(xx2 - xx1, 0.0) * \
                                jnp.maximum(yy2 - yy1, 0.0)
                            carea = (cx2 - cx1) * (cy2 - cy1)
                            union = barea + carea - inter
                            iou = inter / jnp.maximum(union, 1e-9)
                            LST[sl] = jnp.where(iou > _IOU_THR, sentv, il)
                        return 0
                    lax.fori_loop(0, nv4, suppb, 0)

                    sel = (iota16 == r) & alive
                    KX1 = jnp.where(sel, bx1, KX1)
                    KY1 = jnp.where(sel, by1, KY1)
                    KX2 = jnp.where(sel, bx2, KX2)
                    KY2 = jnp.where(sel, by2, KY2)
                    KS = jnp.where(sel, _sp(m, jnp.float32), KS)
                    KM = jnp.where(sel, 1.0, KM)
                    return (KX1, KY1, KX2, KY2, KS, KM)

                z = jnp.zeros((16,), jnp.float32)
                KX1, KY1, KX2, KY2, KS, KM = lax.fori_loop(
                    0, _K, roundb, (z, z, z, z, z, z))
                sb = j * 96
                STG[pl.ds(sb + 0, 16)] = KX1
                STG[pl.ds(sb + 16, 16)] = KY1
                STG[pl.ds(sb + 32, 16)] = KX2
                STG[pl.ds(sb + 48, 16)] = KY2
                STG[pl.ds(sb + 64, 16)] = KS
                STG[pl.ds(sb + 80, 16)] = KM

            pltpu.sync_copy(STG, out_h.at[b, half])

    return k(x1h, y1h, x2h, y2h, sch, clh)


def _pack_body(vr_ref, vc_ref, o_ref):
    vr = vr_ref[0]
    vc = vc_ref[0]
    Mrow = vr[5:6, :]
    ii = lax.broadcasted_iota(jnp.int32, (_MAXD, _MAXD), 0)
    jj = lax.broadcasted_iota(jnp.int32, (_MAXD, _MAXD), 1)
    triu = (ii <= jj).astype(jnp.float32)
    posrow = jnp.dot(Mrow, triu, precision=lax.Precision.HIGHEST,
                     preferred_element_type=jnp.float32)
    posi = posrow.astype(jnp.int32) - 1
    oh = ((ii == posi) & (Mrow > 0.5)).astype(jnp.float32)
    out6 = jnp.dot(oh, vc, precision=lax.Precision.HIGHEST,
                   preferred_element_type=jnp.float32)
    o_ref[0] = out6[:, 0:5]


def kernel(pred_boxes, pred_scores, pred_cls_inds):
    B, N = pred_scores.shape
    NPAD = ((N + _CH - 1) // _CH) * _CH
    pz = ((0, 0), (0, NPAD - N))
    x1 = jnp.pad(pred_boxes[:, :, 0], pz)
    y1 = jnp.pad(pred_boxes[:, :, 1], pz)
    x2 = jnp.pad(pred_boxes[:, :, 2], pz)
    y2 = jnp.pad(pred_boxes[:, :, 3], pz)
    sc = jnp.pad(pred_scores, pz)
    cl = jnp.pad(pred_cls_inds, pz)
    kept = _sc_nms(x1, y1, x2, y2, sc, cl)
    k5 = kept.reshape(B, 2, _K, 6, 16)[:, :, :, :, :_K]
    vrow = k5.transpose(0, 3, 1, 2, 4).reshape(B, 6, _MAXD)
    vcol = vrow.transpose(0, 2, 1)
    out = pl.pallas_call(
        _pack_body,
        grid=(B,),
        in_specs=[
            pl.BlockSpec((1, 6, _MAXD), lambda b: (b, 0, 0)),
            pl.BlockSpec((1, _MAXD, 6), lambda b: (b, 0, 0)),
        ],
        out_specs=pl.BlockSpec((1, _MAXD, 5), lambda b: (b, 0, 0)),
        out_shape=jax.ShapeDtypeStruct((B, _MAXD, 5), jnp.float32),
    )(vrow, vcol)
    return out
